# baseline (speedup 1.0000x reference)
import functools

import jax
import jax.numpy as jnp
from jax import lax
from jax.experimental import pallas as pl
from jax.experimental.pallas import tpu as pltpu
from jax.experimental.pallas import tpu_sc as plsc

NUM_CODONS = 64
EMBED_DIM = 128

_INFO = plsc.get_sparse_core_info()
_NC = _INFO.num_cores
_NS = _INFO.num_subcores
_NW = _NC * _NS
_L = _INFO.num_lanes

_CHUNK = 128
_NSTREAM = 7
_PER_PAIR = _NSTREAM + 1
_BCUTS = {0: (0, 1), 1: (1, 2), 2: (2, 3), 3: (3, 4), 4: (4, 5), 5: (5, 6),
          6: (6, 8)}


def _sc_gather(n_total: int):
  n_chunks = n_total // _CHUNK
  c_per_w = n_chunks // _NW
  n_pairs = c_per_w // _PER_PAIR
  n_g2 = n_pairs // 2
  mesh = plsc.VectorSubcoreMesh(core_axis_name="c", subcore_axis_name="s")

  @functools.partial(
      pl.kernel,
      mesh=mesh,
      out_type=jax.ShapeDtypeStruct((n_chunks, _CHUNK, EMBED_DIM), jnp.float32),
      scratch_types=(
          [pltpu.VMEM((2, _CHUNK, EMBED_DIM), jnp.float32)]
          + [pltpu.VMEM((_CHUNK, EMBED_DIM), jnp.float32)] * 2
          + [pltpu.VMEM((_PER_PAIR, _CHUNK), jnp.int32)] * 2
          + [pltpu.VMEM((NUM_CODONS, EMBED_DIM), jnp.float32)]
          + [pltpu.VMEM_SHARED((NUM_CODONS, EMBED_DIM), jnp.float32)]
          + [pltpu.SemaphoreType.DMA] * (2 + 2 + 2 + 2)
      ),
  )
  def k(idx_hbm, table_hbm, out_hbm, rows, bb0, bb1, ib0, ib1,
        table_v, table_sh, *sems):
    bbufs = (bb0, bb1)
    ibuf = (ib0, ib1)
    sem_g = sems[:2]
    sem_o = sems[2:4]
    sem_i = sems[4:6]
    sem_bo = sems[6:]

    sid = lax.axis_index("s")
    wid = sid * _NC + lax.axis_index("c")
    w_chunk0 = wid * c_per_w

    def idx_fetch(pair, b):
      return pltpu.make_async_copy(
          idx_hbm.at[pl.ds(w_chunk0 + pair * _PER_PAIR, _PER_PAIR)],
          ibuf[b], sem_i[b])

    def gather(j, slot, b):
      return pltpu.make_async_copy(
          table_sh.at[ibuf[b].at[j]], rows.at[slot], sem_g[slot])

    def gather_wait(slot):
      pltpu.make_async_copy(
          table_sh.at[ibuf[0].at[0]], rows.at[slot], sem_g[slot]).wait()

    def out_copy(gc, slot):
      return pltpu.make_async_copy(
          rows.at[slot], out_hbm.at[gc], sem_o[slot])

    def built_out(gc, b2):
      return pltpu.make_async_copy(
          bbufs[b2], out_hbm.at[gc], sem_bo[b2])

    @pl.when(sid == 0)
    def _():
      pltpu.sync_copy(table_hbm, table_sh)
    pltpu.sync_copy(table_hbm, table_v)
    plsc.subcore_barrier()

    idx_fetch(0, 0).start()

    def body(g2, _):
      for u in range(2):
        pair = 2 * g2 + u
        b2 = u

        def bbody(t, _, _u=u, _b2=b2):
          vvec = ibuf[_u][_NSTREAM, pl.ds(t * _L, _L)]
          for n in range(_L):
            v = vvec[n]
            r = t * _L + n
            for db in range(EMBED_DIM // _L):
              bbufs[_b2][r, pl.ds(db * _L, _L)] = (
                  table_v[v, pl.ds(db * _L, _L)])
          return 0

        for j in range(_NSTREAM):
          slot = (u + j) % 2
          gc = w_chunk0 + pair * _PER_PAIR + j

          if u == 0 and j < 2:
            @pl.when(g2 > 0)
            def _():
              out_copy(0, slot).wait()
          else:
            out_copy(0, slot).wait()

          if j == 0:
            idx_fetch(0, u).wait()
            @pl.when(g2 > 0)
            def _():
              built_out(0, b2).wait()

          gather(j, slot, u).start()

          pslot = (u + j - 1) % 2
          if u == 0 and j == 0:
            @pl.when(g2 > 0)
            def _():
              gather_wait(pslot)
              out_copy(gc - 2, pslot).start()
          else:
            gather_wait(pslot)
            out_copy(gc - 1 if j > 0 else gc - 2, pslot).start()

          if j == 0:
            if u == 0:
              idx_fetch(pair + 1, 1 - u).start()
            else:
              @pl.when(g2 < n_g2 - 1)
              def _():
                idx_fetch(pair + 1, 1 - u).start()

          if j in _BCUTS:
            lo, hi = _BCUTS[j]
            lax.fori_loop(lo, hi, bbody, 0)

          if j == _NSTREAM - 1:
            built_out(w_chunk0 + pair * _PER_PAIR + _NSTREAM, b2).start()
      return 0

    lax.fori_loop(0, n_g2, body, 0)

    last_gc = w_chunk0 + (n_pairs - 1) * _PER_PAIR + _NSTREAM - 1
    gather_wait(1)
    out_copy(last_gc, 1).start()
    for slot in range(2):
      out_copy(0, slot).wait()
    for b2 in range(2):
      built_out(0, b2).wait()

  return k


def kernel(x, table):
  batch, seqlen = x.shape
  n_total = batch * seqlen
  idx = x.reshape((n_total // _CHUNK, _CHUNK))
  out = _sc_gather(n_total)(idx, table)
  return out.reshape((batch, seqlen, EMBED_DIM))


# speedup vs baseline: 1.0343x; 1.0343x over previous
import functools

import jax
import jax.numpy as jnp
from jax import lax
from jax.experimental import pallas as pl
from jax.experimental.pallas import tpu as pltpu
from jax.experimental.pallas import tpu_sc as plsc

NUM_CODONS = 64
EMBED_DIM = 128

_INFO = plsc.get_sparse_core_info()
_NC = _INFO.num_cores
_NS = _INFO.num_subcores
_NW = _NC * _NS
_L = _INFO.num_lanes

_CHUNK = 128
_NSTREAM = 7
_PER_PAIR = _NSTREAM + 1
_BCUTS = {0: (0, 2), 1: (2, 4), 2: (4, 6), 3: (6, 8), 4: (8, 10),
          5: (10, 12), 6: (12, 16)}


def _sc_gather(n_total: int):
  n_chunks = n_total // _CHUNK
  c_per_w = n_chunks // _NW
  n_pairs = c_per_w // _PER_PAIR
  n_g4 = n_pairs // 4
  mesh = plsc.VectorSubcoreMesh(core_axis_name="c", subcore_axis_name="s")

  @functools.partial(
      pl.kernel,
      mesh=mesh,
      out_type=jax.ShapeDtypeStruct((n_chunks, _CHUNK, EMBED_DIM), jnp.float32),
      scratch_types=(
          [pltpu.VMEM((4, _CHUNK, EMBED_DIM), jnp.float32)]
          + [pltpu.VMEM((_CHUNK, EMBED_DIM), jnp.float32)] * 2
          + [pltpu.VMEM((_PER_PAIR, _CHUNK), jnp.int32)] * 4
          + [pltpu.VMEM((NUM_CODONS, EMBED_DIM), jnp.float32)]
          + [pltpu.VMEM_SHARED((NUM_CODONS, EMBED_DIM), jnp.float32)]
          + [pltpu.SemaphoreType.DMA] * (4 + 4 + 4 + 2)
      ),
  )
  def k(idx_hbm, table_hbm, out_hbm, rows, bb0, bb1, ib0, ib1, ib2, ib3,
        table_v, table_sh, *sems):
    bbufs = (bb0, bb1)
    ibuf = (ib0, ib1, ib2, ib3)
    sem_g = sems[:4]
    sem_o = sems[4:8]
    sem_i = sems[8:12]
    sem_bo = sems[12:]

    sid = lax.axis_index("s")
    wid = sid * _NC + lax.axis_index("c")
    w_chunk0 = wid * c_per_w

    def idx_fetch(pair, b):
      return pltpu.make_async_copy(
          idx_hbm.at[pl.ds(w_chunk0 + pair * _PER_PAIR, _PER_PAIR)],
          ibuf[b], sem_i[b])

    def gather(j, slot, b):
      return pltpu.make_async_copy(
          table_sh.at[ibuf[b].at[j]], rows.at[slot], sem_g[slot])

    def gather_wait(slot):
      pltpu.make_async_copy(
          table_sh.at[ibuf[0].at[0]], rows.at[slot], sem_g[slot]).wait()

    def out_copy(gc, slot):
      return pltpu.make_async_copy(
          rows.at[slot], out_hbm.at[gc], sem_o[slot])

    def built_out(gc, b2):
      return pltpu.make_async_copy(
          bbufs[b2], out_hbm.at[gc], sem_bo[b2])

    @pl.when(sid == 0)
    def _():
      pltpu.sync_copy(table_hbm, table_sh)
    pltpu.sync_copy(table_hbm, table_v)
    plsc.subcore_barrier()

    idx_fetch(0, 0).start()
    idx_fetch(1, 1).start()

    def body(g4, _):
      for u in range(4):
        pair = 4 * g4 + u
        b2 = u % 2

        def bbody(t, _, _u=u, _b2=b2):
          grp = t >> 1
          dbase = (t & 1) * (EMBED_DIM // 2)
          vvec = ibuf[_u][_NSTREAM, pl.ds(grp * _L, _L)]
          for n in range(_L):
            v = vvec[n]
            r = grp * _L + n
            for db in range(EMBED_DIM // (2 * _L)):
              bbufs[_b2][r, pl.ds(dbase + db * _L, _L)] = (
                  table_v[v, pl.ds(dbase + db * _L, _L)])
          return 0

        for j in range(_NSTREAM):
          slot = (7 * u + j) % 4
          t_c = 7 * pair + j
          gc = w_chunk0 + pair * _PER_PAIR + j

          if u == 0 and j < 4:
            @pl.when(g4 > 0)
            def _():
              out_copy(0, slot).wait()
          else:
            out_copy(0, slot).wait()

          if j == 0:
            idx_fetch(0, u).wait()
            if u < 2:
              @pl.when(g4 > 0)
              def _():
                built_out(0, b2).wait()
            else:
              built_out(0, b2).wait()

          gather(j, slot, u).start()

          pslot = (7 * u + j - 1) % 4
          if u == 0 and j == 0:
            @pl.when(g4 > 0)
            def _():
              gather_wait(pslot)
              out_copy(gc - 2, pslot).start()
          else:
            gather_wait(pslot)
            out_copy(gc - 1 if j > 0 else gc - 2, pslot).start()

          if j == 0:
            if u < 2:
              idx_fetch(pair + 2, (u + 2) % 4).start()
            else:
              @pl.when(g4 < n_g4 - 1)
              def _():
                idx_fetch(pair + 2, (u + 2) % 4).start()

          if j in _BCUTS:
            lo, hi = _BCUTS[j]
            lax.fori_loop(lo, hi, bbody, 0)

          if j == _NSTREAM - 1:
            built_out(w_chunk0 + pair * _PER_PAIR + _NSTREAM, b2).start()
      return 0

    lax.fori_loop(0, n_g4, body, 0)

    last_gc = w_chunk0 + (n_pairs - 1) * _PER_PAIR + _NSTREAM - 1
    gather_wait(3)
    out_copy(last_gc, 3).start()
    for slot in range(4):
      out_copy(0, slot).wait()
    for b2 in range(2):
      built_out(0, b2).wait()

  return k


def kernel(x, table):
  batch, seqlen = x.shape
  n_total = batch * seqlen
  idx = x.reshape((n_total // _CHUNK, _CHUNK))
  out = _sc_gather(n_total)(idx, table)
  return out.reshape((batch, seqlen, EMBED_DIM))
